# Initial kernel scaffold; baseline (speedup 1.0000x reference)
import jax
import jax.numpy as jnp
from jax.experimental import pallas as pl


def kernel(rots, transls, coefs, ts):
    raise NotImplementedError("write your pallas kernel here")


# speedup vs baseline: 1.0054x; 1.0054x over previous
import functools

import jax
import jax.numpy as jnp
import numpy as np
from jax.experimental import pallas as pl
from jax.experimental.pallas import tpu as pltpu

K_B = 32
T_F = 500
N_TS = 8
BG = 4096

_E_NP = np.zeros((96, 96), dtype=np.float32)
for _n in range(N_TS):
    for _q in range(12):
        _E_NP[_q * 8 + _n, _n * 12 + _q] = 1.0


def _mb_kernel(ts_ref, params_ref, eperm_ref, coefs_ref, out_ref, m_ref):
    pi = pl.program_id(0)

    @pl.when(pi == 0)
    def _build_m():
        for n in range(N_TS):
            t = ts_ref[0, n]
            tf = jnp.clip(jnp.floor(t), 0.0, T_F - 1)
            tc = jnp.clip(jnp.ceil(t), 0.0, T_F - 1)
            ipre = tf.astype(jnp.int32)
            inext = tc.astype(jnp.int32)
            w = t - tf
            row_pre = params_ref[ipre]
            row_next = params_ref[inext]
            mixed = (1.0 - w) * row_pre + w * row_next
            for c in range(9):
                m_ref[c * 8 + n, :] = mixed[c, :]

    m = m_ref[:, :]
    cb = coefs_ref[:, :]
    y = jax.lax.dot_general(m, cb, (((1,), (1,)), ((), ())),
                            preferred_element_type=jnp.float32)

    a1x, a1y, a1z = y[0:8], y[8:16], y[16:24]
    a2x, a2y, a2z = y[24:32], y[32:40], y[40:48]
    tx, ty, tz = y[48:56], y[56:64], y[64:72]

    inv1 = 1.0 / jnp.maximum(jnp.sqrt(a1x * a1x + a1y * a1y + a1z * a1z), 1e-12)
    b1x, b1y, b1z = a1x * inv1, a1y * inv1, a1z * inv1

    d = b1x * a2x + b1y * a2y + b1z * a2z
    ux, uy, uz = a2x - d * b1x, a2y - d * b1y, a2z - d * b1z
    inv2 = 1.0 / jnp.maximum(jnp.sqrt(ux * ux + uy * uy + uz * uz), 1e-12)
    b2x, b2y, b2z = ux * inv2, uy * inv2, uz * inv2

    b3x = b1y * b2z - b1z * b2y
    b3y = b1z * b2x - b1x * b2z
    b3z = b1x * b2y - b1y * b2x

    z = jnp.concatenate([
        b1x, b2x, b3x, tx,
        b1y, b2y, b3y, ty,
        b1z, b2z, b3z, tz,
    ], axis=0)

    out_ref[:, :] = jax.lax.dot_general(
        z, eperm_ref[:, :], (((0,), (0,)), ((), ())),
        preferred_element_type=jnp.float32)


@jax.jit
def kernel(rots, transls, coefs, ts):
    G = coefs.shape[0]
    params = jnp.transpose(
        jnp.concatenate([rots, transls], axis=-1), (1, 2, 0))
    ts2 = jnp.reshape(ts, (1, N_TS)).astype(jnp.float32)
    eperm = jnp.asarray(_E_NP)

    grid = (pl.cdiv(G, BG),)
    out = pl.pallas_call(
        _mb_kernel,
        grid=grid,
        in_specs=[
            pl.BlockSpec(memory_space=pltpu.SMEM),
            pl.BlockSpec((T_F, 9, K_B), lambda i: (0, 0, 0)),
            pl.BlockSpec((96, 96), lambda i: (0, 0)),
            pl.BlockSpec((BG, K_B), lambda i: (i, 0)),
        ],
        out_specs=pl.BlockSpec((BG, 96), lambda i: (i, 0)),
        out_shape=jax.ShapeDtypeStruct((G, 96), jnp.float32),
        scratch_shapes=[pltpu.VMEM((72, K_B), jnp.float32)],
    )(ts2, params, eperm, coefs)
    return out.reshape(G, N_TS, 3, 4)


# speedup vs baseline: 5.0605x; 5.0335x over previous
import jax
import jax.numpy as jnp
import numpy as np
from jax.experimental import pallas as pl
from jax.experimental.pallas import tpu as pltpu

K_B = 32
T_F = 500
N_TS = 8
G_PTS = 100000
BG = 5120
NB = 20
N_IN_DMA = 4
N_OUT_DMA = 24

_E_NP = np.zeros((96, 96), dtype=np.float32)
for _n in range(N_TS):
    for _q in range(12):
        _E_NP[_q * 8 + _n, _n * 12 + _q] = 1.0


def _mb_kernel(ts_ref, eperm_ref, params_hbm, coefs_hbm, out_hbm,
               cbuf, obuf, pbuf, m_ref, in_sems, p_sems, out_sems):
    for s in range(N_IN_DMA):
        pltpu.make_async_copy(
            coefs_hbm.at[pl.ds(s * 8, 8), :],
            cbuf.at[pl.ds(s * 8, 8), :],
            in_sems.at[s]).start()

    idx = []
    for n in range(N_TS):
        t = ts_ref[0, n]
        tf = jnp.clip(jnp.floor(t), 0.0, T_F - 1)
        tc = jnp.clip(jnp.ceil(t), 0.0, T_F - 1)
        idx.append((tf.astype(jnp.int32), tc.astype(jnp.int32), t - tf))
    for n in range(N_TS):
        ipre, inext, _ = idx[n]
        pltpu.make_async_copy(params_hbm.at[ipre], pbuf.at[2 * n],
                              p_sems.at[2 * n]).start()
        pltpu.make_async_copy(params_hbm.at[inext], pbuf.at[2 * n + 1],
                              p_sems.at[2 * n + 1]).start()

    for n in range(N_TS):
        ipre, inext, _ = idx[n]
        pltpu.make_async_copy(params_hbm.at[ipre], pbuf.at[2 * n],
                              p_sems.at[2 * n]).wait()
        pltpu.make_async_copy(params_hbm.at[inext], pbuf.at[2 * n + 1],
                              p_sems.at[2 * n + 1]).wait()
        w = idx[n][2]
        mixed = (1.0 - w) * pbuf[2 * n] + w * pbuf[2 * n + 1]
        for c in range(9):
            m_ref[c * 8 + n, :] = mixed[c, :]

    for s in range(N_IN_DMA):
        pltpu.make_async_copy(
            coefs_hbm.at[pl.ds(s * 8, 8), :],
            cbuf.at[pl.ds(s * 8, 8), :],
            in_sems.at[s]).wait()

    m = m_ref[:, :]
    eperm = eperm_ref[:, :]

    off = 0
    for i in range(NB):
        cw = min(BG, G_PTS - off)
        cb = cbuf[:, pl.ds(off, cw)]
        y = jnp.dot(m, cb, preferred_element_type=jnp.float32)

        a1x, a1y, a1z = y[0:8], y[8:16], y[16:24]
        a2x, a2y, a2z = y[24:32], y[32:40], y[40:48]
        tx, ty, tz = y[48:56], y[56:64], y[64:72]

        inv1 = 1.0 / jnp.maximum(
            jnp.sqrt(a1x * a1x + a1y * a1y + a1z * a1z), 1e-12)
        b1x, b1y, b1z = a1x * inv1, a1y * inv1, a1z * inv1

        d = b1x * a2x + b1y * a2y + b1z * a2z
        ux, uy, uz = a2x - d * b1x, a2y - d * b1y, a2z - d * b1z
        inv2 = 1.0 / jnp.maximum(jnp.sqrt(ux * ux + uy * uy + uz * uz), 1e-12)
        b2x, b2y, b2z = ux * inv2, uy * inv2, uz * inv2

        b3x = b1y * b2z - b1z * b2y
        b3y = b1z * b2x - b1x * b2z
        b3z = b1x * b2y - b1y * b2x

        z = jnp.concatenate([
            b1x, b2x, b3x, tx,
            b1y, b2y, b3y, ty,
            b1z, b2z, b3z, tz,
        ], axis=0)

        zp = jax.lax.dot_general(eperm, z, (((0,), (0,)), ((), ())),
                                 preferred_element_type=jnp.float32)
        obuf[:, :, pl.ds(off, cw)] = zp.reshape(24, 4, cw)
        off += cw

    for r in range(N_OUT_DMA):
        pltpu.make_async_copy(obuf.at[r], out_hbm.at[r],
                              out_sems.at[r]).start()
    for r in range(N_OUT_DMA):
        pltpu.make_async_copy(obuf.at[r], out_hbm.at[r],
                              out_sems.at[r]).wait()


@jax.jit
def kernel(rots, transls, coefs, ts):
    G = coefs.shape[0]
    params = jnp.transpose(
        jnp.concatenate([rots, transls], axis=-1), (1, 2, 0))
    ts2 = jnp.reshape(ts, (1, N_TS)).astype(jnp.float32)
    eperm = jnp.asarray(_E_NP)
    coefs_t = coefs.T

    out = pl.pallas_call(
        _mb_kernel,
        in_specs=[
            pl.BlockSpec(memory_space=pltpu.SMEM),
            pl.BlockSpec(memory_space=pltpu.VMEM),
            pl.BlockSpec(memory_space=pltpu.HBM),
            pl.BlockSpec(memory_space=pltpu.HBM),
        ],
        out_specs=pl.BlockSpec(memory_space=pltpu.HBM),
        out_shape=jax.ShapeDtypeStruct((24, 4, G), jnp.float32),
        scratch_shapes=[
            pltpu.VMEM((K_B, G_PTS), jnp.float32),
            pltpu.VMEM((24, 4, G_PTS), jnp.float32),
            pltpu.VMEM((16, 9, K_B), jnp.float32),
            pltpu.VMEM((72, K_B), jnp.float32),
            pltpu.SemaphoreType.DMA((N_IN_DMA,)),
            pltpu.SemaphoreType.DMA((16,)),
            pltpu.SemaphoreType.DMA((N_OUT_DMA,)),
        ],
    )(ts2, eperm, params, coefs_t)
    return jnp.transpose(out.reshape(N_TS, 3, 4, G), (3, 0, 1, 2))


# speedup vs baseline: 7.7333x; 1.5282x over previous
import jax
import jax.numpy as jnp
import numpy as np
from jax.experimental import pallas as pl
from jax.experimental.pallas import tpu as pltpu

K_B = 32
T_F = 500
N_TS = 8
G_PTS = 100000
G_PAD = 100096
BG = 5120
NB = 20
N_IN_DMA = 4

_E_NP = np.zeros((96, 96), dtype=np.float32)
for _n in range(N_TS):
    for _q in range(12):
        _E_NP[_q * 8 + _n, _n * 12 + _q] = 1.0


def _mb_kernel(ts_ref, eperm_ref, params_hbm, coefs_hbm, out_hbm,
               cbuf, obuf, pbuf, m_ref, in_sems, p_sems, out_sems):
    for s in range(N_IN_DMA):
        pltpu.make_async_copy(
            coefs_hbm.at[pl.ds(s * 8, 8), :],
            cbuf.at[pl.ds(s * 8, 8), :],
            in_sems.at[s]).start()

    idx = []
    for n in range(N_TS):
        t = ts_ref[0, n]
        tf = jnp.clip(jnp.floor(t), 0.0, T_F - 1)
        tc = jnp.clip(jnp.ceil(t), 0.0, T_F - 1)
        idx.append((tf.astype(jnp.int32), tc.astype(jnp.int32), t - tf))
    for n in range(N_TS):
        ipre, inext, _ = idx[n]
        pltpu.make_async_copy(params_hbm.at[ipre], pbuf.at[2 * n],
                              p_sems.at[2 * n]).start()
        pltpu.make_async_copy(params_hbm.at[inext], pbuf.at[2 * n + 1],
                              p_sems.at[2 * n + 1]).start()

    for n in range(N_TS):
        ipre, inext, w = idx[n]
        pltpu.make_async_copy(params_hbm.at[ipre], pbuf.at[2 * n],
                              p_sems.at[2 * n]).wait()
        pltpu.make_async_copy(params_hbm.at[inext], pbuf.at[2 * n + 1],
                              p_sems.at[2 * n + 1]).wait()
        mixed = (1.0 - w) * pbuf[2 * n] + w * pbuf[2 * n + 1]
        for c in range(9):
            m_ref[c * 8 + n, :] = mixed[c, :]

    for s in range(N_IN_DMA):
        pltpu.make_async_copy(
            coefs_hbm.at[pl.ds(s * 8, 8), :],
            cbuf.at[pl.ds(s * 8, 8), :],
            in_sems.at[s]).wait()

    m = m_ref[:, :]
    eperm = eperm_ref[:, :]

    def out_copy(i):
        off = i * BG
        cwp = min(BG, G_PAD - off)
        return pltpu.make_async_copy(
            obuf.at[:, :, pl.ds(off, cwp)],
            out_hbm.at[:, :, pl.ds(off, cwp)],
            out_sems.at[i])

    for i in range(NB):
        off = i * BG
        cw = min(BG, G_PTS - off)
        cb = cbuf[:, pl.ds(off, cw)]
        y = jnp.dot(m, cb, preferred_element_type=jnp.float32)

        a1x, a1y, a1z = y[0:8], y[8:16], y[16:24]
        a2x, a2y, a2z = y[24:32], y[32:40], y[40:48]
        tx, ty, tz = y[48:56], y[56:64], y[64:72]

        inv1 = jax.lax.rsqrt(
            jnp.maximum(a1x * a1x + a1y * a1y + a1z * a1z, 1e-24))
        b1x, b1y, b1z = a1x * inv1, a1y * inv1, a1z * inv1

        d = b1x * a2x + b1y * a2y + b1z * a2z
        ux, uy, uz = a2x - d * b1x, a2y - d * b1y, a2z - d * b1z
        inv2 = jax.lax.rsqrt(jnp.maximum(ux * ux + uy * uy + uz * uz, 1e-24))
        b2x, b2y, b2z = ux * inv2, uy * inv2, uz * inv2

        b3x = b1y * b2z - b1z * b2y
        b3y = b1z * b2x - b1x * b2z
        b3z = b1x * b2y - b1y * b2x

        z = jnp.concatenate([
            b1x, b2x, b3x, tx,
            b1y, b2y, b3y, ty,
            b1z, b2z, b3z, tz,
        ], axis=0)

        zp = jax.lax.dot_general(eperm, z, (((0,), (0,)), ((), ())),
                                 preferred_element_type=jnp.float32)
        obuf[:, :, pl.ds(off, cw)] = zp.reshape(24, 4, cw)
        out_copy(i).start()

    for i in range(NB):
        out_copy(i).wait()


@jax.jit
def kernel(rots, transls, coefs, ts):
    G = coefs.shape[0]
    params = jnp.transpose(
        jnp.concatenate([rots, transls], axis=-1), (1, 2, 0))
    ts2 = jnp.reshape(ts, (1, N_TS)).astype(jnp.float32)
    eperm = jnp.asarray(_E_NP)
    coefs_t = coefs.T

    out = pl.pallas_call(
        _mb_kernel,
        in_specs=[
            pl.BlockSpec(memory_space=pltpu.SMEM),
            pl.BlockSpec(memory_space=pltpu.VMEM),
            pl.BlockSpec(memory_space=pltpu.HBM),
            pl.BlockSpec(memory_space=pltpu.HBM),
        ],
        out_specs=pl.BlockSpec(memory_space=pltpu.HBM),
        out_shape=jax.ShapeDtypeStruct((24, 4, G_PAD), jnp.float32),
        scratch_shapes=[
            pltpu.VMEM((K_B, G_PTS), jnp.float32),
            pltpu.VMEM((24, 4, G_PAD), jnp.float32),
            pltpu.VMEM((16, 9, K_B), jnp.float32),
            pltpu.VMEM((72, K_B), jnp.float32),
            pltpu.SemaphoreType.DMA((N_IN_DMA,)),
            pltpu.SemaphoreType.DMA((16,)),
            pltpu.SemaphoreType.DMA((NB,)),
        ],
    )(ts2, eperm, params, coefs_t)
    return jnp.transpose(out.reshape(N_TS, 3, 4, G_PAD), (3, 0, 1, 2))[:G]
